# baseline (speedup 1.0000x reference)
import functools

import jax
import jax.numpy as jnp
from jax import lax
from jax.experimental import pallas as pl
from jax.experimental.pallas import tpu as pltpu
from jax.experimental.pallas import tpu_sc as plsc

_B = 1024
_N_GENES = 20000
_N_COMP = 64

_NC = 2
_NS = 16
_NW = _NC * _NS
_BPW = _B // _NW


def _height_body(table_hbm, idx_hbm, lat_hbm, out_hbm, idx_v, lat_v, rows_v, sem):
    wid = lax.axis_index("s") * _NC + lax.axis_index("c")
    base = wid * _BPW
    pltpu.sync_copy(idx_hbm.at[pl.ds(base, _BPW)], idx_v)
    pltpu.sync_copy(lat_hbm.at[pl.ds(base, _BPW)], lat_v)
    pltpu.async_copy(table_hbm.at[idx_v], rows_v, sem).wait()
    for g in range(_BPW // 16):
        lat16 = lat_v[pl.ds(g * 16, 16)]
        for b_local in range(16):
            b = g * 16 + b_local
            lat_b = lat16[b_local]
            for j in range(_N_COMP // 16):
                sl = pl.ds(j * 16, 16)
                rows_v[b, 0, sl] = rows_v[b, 0, sl] * lat_b
    pltpu.sync_copy(rows_v, out_hbm.at[pl.ds(base, _BPW)])


@functools.cache
def _height_sc():
    return pl.kernel(
        _height_body,
        mesh=plsc.VectorSubcoreMesh(core_axis_name="c", subcore_axis_name="s",
                                    num_cores=_NC, num_subcores=_NS),
        out_type=jax.ShapeDtypeStruct((_B, 1, _N_COMP), jnp.float32),
        scratch_types=[
            pltpu.VMEM((_BPW,), jnp.int32),
            pltpu.VMEM((_BPW,), jnp.float32),
            pltpu.VMEM((_BPW, 1, _N_COMP), jnp.float32),
            pltpu.SemaphoreType.DMA,
        ],
        compiler_params=pltpu.CompilerParams(use_tc_tiling_on_sc=False),
    )


_GB = 1250
_NCHUNK = _N_GENES // _GB
_NBUF = 4


def _outer_body(w_ref, lat_ref, out_hbm, buf, sems):
    i = pl.program_id(0)
    slot = lax.rem(i, _NBUF)
    for s in range(_NBUF):
        @pl.when(jnp.logical_and(slot == s, i >= _NBUF))
        def _():
            prev = i - _NBUF
            pltpu.make_async_copy(
                buf.at[s], out_hbm.at[pl.ds(prev * _GB, _GB)], sems.at[s]
            ).wait()
        @pl.when(slot == s)
        def _():
            prod = lax.dot_general(
                w_ref[...].reshape(1, _GB), lat_ref[...],
                (((0,), (0,)), ((), ())),
                preferred_element_type=jnp.float32,
                precision=lax.Precision.HIGHEST,
            )
            buf[s] = prod.reshape(_GB, 1, _B)
            pltpu.make_async_copy(
                buf.at[s], out_hbm.at[pl.ds(i * _GB, _GB)], sems.at[s]
            ).start()
    @pl.when(i == _NCHUNK - 1)
    def _():
        for k in range(_NBUF):
            c = _NCHUNK - _NBUF + k
            pltpu.make_async_copy(
                buf.at[c % _NBUF], out_hbm.at[pl.ds(c * _GB, _GB)],
                sems.at[c % _NBUF],
            ).wait()


def _overall_tc(w3, lat3):
    return pl.pallas_call(
        _outer_body,
        grid=(_NCHUNK,),
        in_specs=[
            pl.BlockSpec((1, 1, _GB), lambda i: (i, 0, 0)),
            pl.BlockSpec((1, _B), lambda i: (0, 0)),
        ],
        out_specs=pl.BlockSpec(memory_space=pl.ANY),
        out_shape=jax.ShapeDtypeStruct((_N_GENES, 1, _B), jnp.float32),
        scratch_shapes=[
            pltpu.VMEM((_NBUF, _GB, 1, _B), jnp.float32),
            pltpu.SemaphoreType.DMA((_NBUF,)),
        ],
    )(w3, lat3)


def kernel(latent, genes_oi, height_weight, overall_weight):
    lat = latent.reshape(_B)
    height3d = _height_sc()(height_weight, genes_oi, lat)
    out3 = _overall_tc(overall_weight.reshape(_NCHUNK, 1, _GB),
                       latent.reshape(1, _B))
    overall = out3.transpose(2, 0, 1)
    return (height3d, overall)


# speedup vs baseline: 1.0838x; 1.0838x over previous
import functools

import jax
import jax.numpy as jnp
from jax import lax
from jax.experimental import pallas as pl
from jax.experimental.pallas import tpu as pltpu
from jax.experimental.pallas import tpu_sc as plsc

_B = 1024
_N_GENES = 20000
_N_COMP = 64

_NC = 2
_NS = 16
_NW = _NC * _NS
_BPW = _B // _NW


def _height_body(table_hbm, idx_hbm, lat_hbm, out_hbm, idx_v, lat_v, rows_v, sem):
    wid = lax.axis_index("s") * _NC + lax.axis_index("c")
    base = wid * _BPW
    pltpu.sync_copy(idx_hbm.at[pl.ds(base, _BPW)], idx_v)
    pltpu.sync_copy(lat_hbm.at[pl.ds(base, _BPW)], lat_v)
    pltpu.async_copy(table_hbm.at[idx_v], rows_v, sem).wait()
    for g in range(_BPW // 16):
        lat16 = lat_v[pl.ds(g * 16, 16)]
        for b_local in range(16):
            b = g * 16 + b_local
            lat_b = lat16[b_local]
            for j in range(_N_COMP // 16):
                sl = pl.ds(j * 16, 16)
                rows_v[b, 0, sl] = rows_v[b, 0, sl] * lat_b
    pltpu.sync_copy(rows_v, out_hbm.at[pl.ds(base, _BPW)])


@functools.cache
def _height_sc():
    return pl.kernel(
        _height_body,
        mesh=plsc.VectorSubcoreMesh(core_axis_name="c", subcore_axis_name="s",
                                    num_cores=_NC, num_subcores=_NS),
        out_type=jax.ShapeDtypeStruct((_B, 1, _N_COMP), jnp.float32),
        scratch_types=[
            pltpu.VMEM((_BPW,), jnp.int32),
            pltpu.VMEM((_BPW,), jnp.float32),
            pltpu.VMEM((_BPW, 1, _N_COMP), jnp.float32),
            pltpu.SemaphoreType.DMA,
        ],
        compiler_params=pltpu.CompilerParams(use_tc_tiling_on_sc=False),
    )


_GB = 1250
_NCHUNK = _N_GENES // _GB
_NBUF = 4


def _outer_body(w_ref, lat_ref, out_hbm, buf, sems):
    i = pl.program_id(0)
    slot = lax.rem(i, _NBUF)
    for s in range(_NBUF):
        @pl.when(jnp.logical_and(slot == s, i >= _NBUF))
        def _():
            prev = i - _NBUF
            pltpu.make_async_copy(
                buf.at[s], out_hbm.at[pl.ds(prev * _GB, _GB), 0], sems.at[s]
            ).wait()
        @pl.when(slot == s)
        def _():
            prod = lax.dot_general(
                w_ref[...].reshape(1, _GB), lat_ref[...],
                (((0,), (0,)), ((), ())),
                preferred_element_type=jnp.float32,
                precision=lax.Precision.HIGHEST,
            )
            buf[s] = prod
            pltpu.make_async_copy(
                buf.at[s], out_hbm.at[pl.ds(i * _GB, _GB), 0], sems.at[s]
            ).start()
    @pl.when(i == _NCHUNK - 1)
    def _():
        for k in range(_NBUF):
            c = _NCHUNK - _NBUF + k
            pltpu.make_async_copy(
                buf.at[c % _NBUF], out_hbm.at[pl.ds(c * _GB, _GB), 0],
                sems.at[c % _NBUF],
            ).wait()


def _overall_tc(w3, lat3):
    return pl.pallas_call(
        _outer_body,
        grid=(_NCHUNK,),
        in_specs=[
            pl.BlockSpec((1, 1, _GB), lambda i: (i, 0, 0)),
            pl.BlockSpec((1, _B), lambda i: (0, 0)),
        ],
        out_specs=pl.BlockSpec(memory_space=pl.ANY),
        out_shape=jax.ShapeDtypeStruct((_N_GENES, 1, _B), jnp.float32),
        scratch_shapes=[
            pltpu.VMEM((_NBUF, _GB, _B), jnp.float32),
            pltpu.SemaphoreType.DMA((_NBUF,)),
        ],
    )(w3, lat3)


def kernel(latent, genes_oi, height_weight, overall_weight):
    lat = latent.reshape(_B)
    height3d = _height_sc()(height_weight, genes_oi, lat)
    out3 = _overall_tc(overall_weight.reshape(_NCHUNK, 1, _GB),
                       latent.reshape(1, _B))
    overall = out3.transpose(2, 0, 1)
    return (height3d, overall)


# speedup vs baseline: 1.7579x; 1.6220x over previous
import functools

import jax
import jax.numpy as jnp
from jax import lax
from jax.experimental import pallas as pl
from jax.experimental.pallas import tpu as pltpu
from jax.experimental.pallas import tpu_sc as plsc

_B = 1024
_N_GENES = 20000
_N_COMP = 64

_NC = 2
_NS = 16
_NW = _NC * _NS
_BPW = _B // _NW


def _height_body(table_hbm, idx_hbm, lat_hbm, out_hbm, idx_v, lat_v, rows_v, sem):
    wid = lax.axis_index("s") * _NC + lax.axis_index("c")
    base = wid * _BPW
    pltpu.sync_copy(idx_hbm.at[pl.ds(base, _BPW)], idx_v)
    pltpu.sync_copy(lat_hbm.at[pl.ds(base, _BPW)], lat_v)
    pltpu.async_copy(table_hbm.at[idx_v], rows_v, sem).wait()
    for g in range(_BPW // 16):
        lat16 = lat_v[pl.ds(g * 16, 16)]
        for b_local in range(16):
            b = g * 16 + b_local
            lat_b = lat16[b_local]
            for j in range(_N_COMP // 16):
                sl = pl.ds(j * 16, 16)
                rows_v[b, 0, sl] = rows_v[b, 0, sl] * lat_b
    pltpu.sync_copy(rows_v, out_hbm.at[pl.ds(base, _BPW)])


@functools.cache
def _height_sc():
    return pl.kernel(
        _height_body,
        mesh=plsc.VectorSubcoreMesh(core_axis_name="c", subcore_axis_name="s",
                                    num_cores=_NC, num_subcores=_NS),
        out_type=jax.ShapeDtypeStruct((_B, 1, _N_COMP), jnp.float32),
        scratch_types=[
            pltpu.VMEM((_BPW,), jnp.int32),
            pltpu.VMEM((_BPW,), jnp.float32),
            pltpu.VMEM((_BPW, 1, _N_COMP), jnp.float32),
            pltpu.SemaphoreType.DMA,
        ],
        compiler_params=pltpu.CompilerParams(use_tc_tiling_on_sc=False),
    )


_GB = 1250
_NCHUNK = _N_GENES // _GB
_NBUF = 4


def _outer_body(w_ref, lat_ref, out_hbm, buf, sems):
    i = pl.program_id(0)
    slot = lax.rem(i, _NBUF)
    for s in range(_NBUF):
        @pl.when(jnp.logical_and(slot == s, i >= _NBUF))
        def _():
            prev = i - _NBUF
            pltpu.make_async_copy(
                buf.at[s], out_hbm.at[pl.ds(prev * _GB, _GB)], sems.at[s]
            ).wait()
        @pl.when(slot == s)
        def _():
            wt = jnp.transpose(w_ref[...].reshape(1, _GB))
            buf[s] = wt.reshape(_GB, 1, 1) * lat_ref[...].reshape(1, 1, _B)
            pltpu.make_async_copy(
                buf.at[s], out_hbm.at[pl.ds(i * _GB, _GB)], sems.at[s]
            ).start()
    @pl.when(i == _NCHUNK - 1)
    def _():
        for k in range(_NBUF):
            c = _NCHUNK - _NBUF + k
            pltpu.make_async_copy(
                buf.at[c % _NBUF], out_hbm.at[pl.ds(c * _GB, _GB)],
                sems.at[c % _NBUF],
            ).wait()


def _overall_tc(w3, lat3):
    return pl.pallas_call(
        _outer_body,
        grid=(_NCHUNK,),
        in_specs=[
            pl.BlockSpec((1, 1, _GB), lambda i: (i, 0, 0)),
            pl.BlockSpec((1, _B), lambda i: (0, 0)),
        ],
        out_specs=pl.BlockSpec(memory_space=pl.ANY),
        out_shape=jax.ShapeDtypeStruct((_N_GENES, 1, _B), jnp.float32),
        scratch_shapes=[
            pltpu.VMEM((_NBUF, _GB, 1, _B), jnp.float32),
            pltpu.SemaphoreType.DMA((_NBUF,)),
        ],
    )(w3, lat3)


def kernel(latent, genes_oi, height_weight, overall_weight):
    lat = latent.reshape(_B)
    height3d = _height_sc()(height_weight, genes_oi, lat)
    out3 = _overall_tc(overall_weight.reshape(_NCHUNK, 1, _GB),
                       latent.reshape(1, _B))
    overall = out3.transpose(2, 0, 1)
    return (height3d, overall)


# speedup vs baseline: 2.1509x; 1.2236x over previous
import functools

import jax
import jax.numpy as jnp
from jax import lax
from jax.experimental import pallas as pl
from jax.experimental.pallas import tpu as pltpu
from jax.experimental.pallas import tpu_sc as plsc

_B = 1024
_N_GENES = 20000
_N_COMP = 64

_NC = 2
_NS = 16
_NW = _NC * _NS
_BPW = _B // _NW


def _height_body(tflat_hbm, idx_hbm, lat_hbm, out_hbm, idx_v, lat_v, idx2, buf, sem):
    wid = lax.axis_index("s") * _NC + lax.axis_index("c")
    base = wid * _BPW
    pltpu.sync_copy(idx_hbm.at[pl.ds(base, _BPW)], idx_v)
    pltpu.sync_copy(lat_hbm.at[pl.ds(base, _BPW)], lat_v)
    for c in range(_N_COMP):
        for h in range(_BPW // 16):
            sl = pl.ds(h * 16, 16)
            idx2[c, sl] = idx_v[sl] + (c * _N_GENES)
    copies = [
        pltpu.async_copy(tflat_hbm.at[idx2.at[c]], buf.at[c], sem)
        for c in range(_N_COMP)
    ]
    for cp in copies:
        cp.wait()
    for c in range(_N_COMP):
        for h in range(_BPW // 16):
            sl = pl.ds(h * 16, 16)
            buf[c, sl] = buf[c, sl] * lat_v[sl]
    pltpu.sync_copy(buf, out_hbm.at[:, pl.ds(base, _BPW)])


@functools.cache
def _height_sc():
    return pl.kernel(
        _height_body,
        mesh=plsc.VectorSubcoreMesh(core_axis_name="c", subcore_axis_name="s",
                                    num_cores=_NC, num_subcores=_NS),
        out_type=jax.ShapeDtypeStruct((_N_COMP, _B), jnp.float32),
        scratch_types=[
            pltpu.VMEM((_BPW,), jnp.int32),
            pltpu.VMEM((_BPW,), jnp.float32),
            pltpu.VMEM((_N_COMP, _BPW), jnp.int32),
            pltpu.VMEM((_N_COMP, _BPW), jnp.float32),
            pltpu.SemaphoreType.DMA,
        ],
        compiler_params=pltpu.CompilerParams(use_tc_tiling_on_sc=False),
    )


_GB = 1250
_NCHUNK = _N_GENES // _GB
_NBUF = 4


def _outer_body(w_ref, lat_ref, out_hbm, buf, sems):
    i = pl.program_id(0)
    slot = lax.rem(i, _NBUF)
    for s in range(_NBUF):
        @pl.when(jnp.logical_and(slot == s, i >= _NBUF))
        def _():
            prev = i - _NBUF
            pltpu.make_async_copy(
                buf.at[s], out_hbm.at[pl.ds(prev * _GB, _GB)], sems.at[s]
            ).wait()
        @pl.when(slot == s)
        def _():
            wt = jnp.transpose(w_ref[...].reshape(1, _GB))
            buf[s] = wt.reshape(_GB, 1, 1) * lat_ref[...].reshape(1, 1, _B)
            pltpu.make_async_copy(
                buf.at[s], out_hbm.at[pl.ds(i * _GB, _GB)], sems.at[s]
            ).start()
    @pl.when(i == _NCHUNK - 1)
    def _():
        for k in range(_NBUF):
            c = _NCHUNK - _NBUF + k
            pltpu.make_async_copy(
                buf.at[c % _NBUF], out_hbm.at[pl.ds(c * _GB, _GB)],
                sems.at[c % _NBUF],
            ).wait()


def _overall_tc(w3, lat3):
    return pl.pallas_call(
        _outer_body,
        grid=(_NCHUNK,),
        in_specs=[
            pl.BlockSpec((1, 1, _GB), lambda i: (i, 0, 0)),
            pl.BlockSpec((1, _B), lambda i: (0, 0)),
        ],
        out_specs=pl.BlockSpec(memory_space=pl.ANY),
        out_shape=jax.ShapeDtypeStruct((_N_GENES, 1, _B), jnp.float32),
        scratch_shapes=[
            pltpu.VMEM((_NBUF, _GB, 1, _B), jnp.float32),
            pltpu.SemaphoreType.DMA((_NBUF,)),
        ],
    )(w3, lat3)


def kernel(latent, genes_oi, height_weight, overall_weight):
    lat = latent.reshape(_B)
    tflat = height_weight.transpose(1, 2, 0).reshape(_N_COMP * _N_GENES)
    height_t = _height_sc()(tflat, genes_oi, lat)
    height3d = height_t.reshape(1, _N_COMP, _B).transpose(2, 0, 1)
    out3 = _overall_tc(overall_weight.reshape(_NCHUNK, 1, _GB),
                       latent.reshape(1, _B))
    overall = out3.transpose(2, 0, 1)
    return (height3d, overall)
